# Initial kernel scaffold; baseline (speedup 1.0000x reference)
import jax
import jax.numpy as jnp
from jax.experimental import pallas as pl


def kernel(x, Wr, br, W1, b1, W2, b2):
    raise NotImplementedError("write your pallas kernel here")


# speedup vs baseline: 1.0619x; 1.0619x over previous
import jax
import jax.numpy as jnp
from jax.experimental import pallas as pl
from jax.experimental.pallas import tpu as pltpu

_NUM_EXPERTS = 8
_D_MODEL = 1024
_D_FF = 2048
_D_OUT = 1024
_N_TOK = 4096
_TB = 512
_NT = _N_TOK // _TB


def _moe_dense_body(x_ref, Wr_ref, br_ref, W1_ref, b1_ref, W2_ref, b2_ref,
                    out_ref, probs_ref, counts_ref):
    t = pl.program_id(0)
    e = pl.program_id(1)

    xb = x_ref[...]

    @pl.when(e == 0)
    def _router():
        logits = jnp.dot(xb, Wr_ref[...], preferred_element_type=jnp.float32)
        logits = logits + br_ref[...][None, :]
        m = jnp.max(logits, axis=-1, keepdims=True)
        unn = jnp.exp(logits - m)
        probs = unn / jnp.sum(unn, axis=-1, keepdims=True)
        probs_ref[...] = probs

        routes = jnp.argmax(probs, axis=-1)
        onehot = (routes[:, None] == jax.lax.broadcasted_iota(
            jnp.int32, (1, _NUM_EXPERTS), 1)).astype(jnp.float32)
        blk_counts = jnp.sum(onehot, axis=0, keepdims=True)
        prev = jnp.where(t == 0, 0.0, counts_ref[...])
        counts_ref[...] = prev + blk_counts

    routes = jnp.argmax(probs_ref[...], axis=-1)
    mask = (routes == e)[:, None]

    h = jnp.maximum(
        jnp.dot(xb, W1_ref[0], preferred_element_type=jnp.float32)
        + b1_ref[0, 0][None, :], 0.0)
    y = jnp.dot(h, W2_ref[0], preferred_element_type=jnp.float32) \
        + b2_ref[0, 0][None, :]

    prev = jnp.where(e == 0, 0.0, out_ref[...])
    out_ref[...] = jnp.where(mask, y, prev)


def kernel(x, Wr, br, W1, b1, W2, b2):
    out, probs, counts = pl.pallas_call(
        _moe_dense_body,
        grid=(_NT, _NUM_EXPERTS),
        in_specs=[
            pl.BlockSpec((_TB, _D_MODEL), lambda t, e: (t, 0)),
            pl.BlockSpec((_D_MODEL, _NUM_EXPERTS), lambda t, e: (0, 0)),
            pl.BlockSpec((_NUM_EXPERTS,), lambda t, e: (0,)),
            pl.BlockSpec((1, _D_MODEL, _D_FF), lambda t, e: (e, 0, 0)),
            pl.BlockSpec((1, 1, _D_FF), lambda t, e: (e, 0, 0)),
            pl.BlockSpec((1, _D_FF, _D_OUT), lambda t, e: (e, 0, 0)),
            pl.BlockSpec((1, 1, _D_OUT), lambda t, e: (e, 0, 0)),
        ],
        out_specs=[
            pl.BlockSpec((_TB, _D_OUT), lambda t, e: (t, 0)),
            pl.BlockSpec((_TB, _NUM_EXPERTS), lambda t, e: (t, 0)),
            pl.BlockSpec((1, _NUM_EXPERTS), lambda t, e: (0, 0)),
        ],
        out_shape=[
            jax.ShapeDtypeStruct((_N_TOK, _D_OUT), jnp.float32),
            jax.ShapeDtypeStruct((_N_TOK, _NUM_EXPERTS), jnp.float32),
            jax.ShapeDtypeStruct((1, _NUM_EXPERTS), jnp.float32),
        ],
        compiler_params=pltpu.CompilerParams(
            dimension_semantics=("arbitrary", "arbitrary"),
        ),
    )(x, Wr, br, W1,
      b1.reshape(_NUM_EXPERTS, 1, _D_FF), W2,
      b2.reshape(_NUM_EXPERTS, 1, _D_OUT))
    return out, probs, counts.reshape(_NUM_EXPERTS)


# speedup vs baseline: 2.4401x; 2.2978x over previous
import functools

import jax
import jax.numpy as jnp
from jax import lax
from jax.experimental import pallas as pl
from jax.experimental.pallas import tpu as pltpu
from jax.experimental.pallas import tpu_sc as plsc

_E = 8
_D = 1024
_F = 2048
_O = 1024
_N = 4096

_TB = 512
_NB = _N // _TB
_MB = 256
_NT = _N // _MB
_NW = _NT + _E - 1
_WPAD = 32

_NWORK = 32
_RPW = _N // _NWORK
_CH = 64


def _router_body(x_ref, Wr_ref, br_ref, probs_ref, ranks_ref, counts_ref):
    b = pl.program_id(0)
    xb = x_ref[...]
    logits = jnp.dot(xb, Wr_ref[...], preferred_element_type=jnp.float32)
    logits = logits + br_ref[...][None, :]
    m = jnp.max(logits, axis=-1, keepdims=True)
    unn = jnp.exp(logits - m)
    probs = unn / jnp.sum(unn, axis=-1, keepdims=True)
    probs_ref[...] = probs

    routes = jnp.argmax(probs, axis=-1)
    iota_e = lax.broadcasted_iota(jnp.int32, (1, _E), 1)
    onehot = (routes[:, None] == iota_e).astype(jnp.float32)

    r = lax.broadcasted_iota(jnp.int32, (_TB, _TB), 0)
    c = lax.broadcasted_iota(jnp.int32, (_TB, _TB), 1)
    tril = (c < r).astype(jnp.float32)
    ranks_in = jnp.dot(tril, onehot, preferred_element_type=jnp.float32)

    prev = jnp.where(b == 0, 0.0, counts_ref[...])
    ranks_ref[...] = ranks_in + prev
    counts_ref[...] = prev + jnp.sum(onehot, axis=0, keepdims=True)


def _col_prefix(row, strict):
    a = jnp.broadcast_to(row, (_E, _E))
    r = lax.broadcasted_iota(jnp.int32, (_E, _E), 0)
    c = lax.broadcasted_iota(jnp.int32, (_E, _E), 1)
    sel = (c < r) if strict else (c <= r)
    return jnp.sum(jnp.where(sel, a, 0), axis=1, keepdims=True)


def _row_from_col(col):
    a = jnp.broadcast_to(col, (_E, _E))
    r = lax.broadcasted_iota(jnp.int32, (_E, _E), 0)
    c = lax.broadcasted_iota(jnp.int32, (_E, _E), 1)
    return jnp.sum(jnp.where(r == c, a, 0), axis=0, keepdims=True)


def _slots_body(probs_ref, ranks_ref, counts_ref, slots_ref, meta_ref):
    b = pl.program_id(0)
    counts_row = counts_ref[...]

    off_start_col = _col_prefix(counts_row, strict=True)
    off_start_row = _row_from_col(off_start_col)

    probs = probs_ref[...]
    routes = jnp.argmax(probs, axis=-1)
    iota_e = lax.broadcasted_iota(jnp.int32, (1, _E), 1)
    onehot = (routes[:, None] == iota_e).astype(jnp.float32)
    slot = jnp.sum(onehot * (off_start_row + ranks_ref[...]),
                   axis=1, keepdims=True)
    slots_ref[...] = slot.astype(jnp.int32)

    @pl.when(b == 0)
    def _meta():
        cnt_col = _col_prefix(counts_row, strict=False) - off_start_col
        cnt_i = cnt_col.astype(jnp.int32)
        os_i = off_start_col.astype(jnp.int32)
        oe_i = os_i + cnt_i
        ft = os_i // _MB
        lte = (oe_i + (_MB - 1)) // _MB
        n_tiles = jnp.where(cnt_i > 0, lte - ft, 0)

        nt_row = _row_from_col(n_tiles)
        item_start = _col_prefix(nt_row, strict=True)
        item_end = item_start + n_tiles
        n_items = jnp.sum(n_tiles)

        w = lax.broadcasted_iota(jnp.int32, (1, _WPAD), 1)
        m8 = (item_start <= w) & (w < item_end)
        e_iota = lax.broadcasted_iota(jnp.int32, (_E, _WPAD), 0)

        def sel(col):
            return jnp.sum(jnp.where(m8, jnp.broadcast_to(col, (_E, _WPAD)),
                                     0), axis=0, keepdims=True)

        e_of_w = jnp.sum(jnp.where(m8, e_iota, 0), axis=0, keepdims=True)
        j_of_w = w - sel(item_start)
        t_of_w = sel(ft) + j_of_w
        lo_w = jnp.maximum(sel(os_i), t_of_w * _MB)
        hi_w = jnp.minimum(sel(oe_i), (t_of_w + 1) * _MB)

        valid = w < n_items
        wi_tile = jnp.where(valid, t_of_w, _NT - 1)
        wi_expert = jnp.where(valid, e_of_w, _E - 1)
        wi_lo = jnp.where(valid, lo_w, _N)
        wi_hi = jnp.where(valid, hi_w, _N)
        zeros = jnp.zeros((4, _WPAD), jnp.int32)
        meta_ref[...] = jnp.concatenate(
            [wi_tile, wi_expert, wi_lo, wi_hi, zeros], axis=0)


def _mlp_body(meta_ref, xs_ref, W1_ref, b1_ref, W2_ref, b2_ref, ys_ref):
    w = pl.program_id(0)
    tile = meta_ref[0, w]
    lo = meta_ref[2, w]
    hi = meta_ref[3, w]
    rows = tile * _MB + lax.broadcasted_iota(jnp.int32, (_MB, 1), 0)
    mask = (rows >= lo) & (rows < hi)

    xb = xs_ref[...]
    h = jnp.maximum(
        jnp.dot(xb, W1_ref[0], preferred_element_type=jnp.float32)
        + b1_ref[0, 0][None, :], 0.0)
    y = jnp.dot(h, W2_ref[0], preferred_element_type=jnp.float32) \
        + b2_ref[0, 0][None, :]

    t_prev = jnp.where(w == 0, -1, meta_ref[0, jnp.maximum(w - 1, 0)])
    prev = jnp.where(tile != t_prev, 0.0, ys_ref[...])
    ys_ref[...] = jnp.where(mask, y, prev)


def _sc_mesh():
    return plsc.VectorSubcoreMesh(core_axis_name="c", subcore_axis_name="s")


def _scatter_rows(x, slots):
    @functools.partial(
        pl.kernel, mesh=_sc_mesh(),
        out_type=jax.ShapeDtypeStruct((_N, _D), jnp.float32),
        scratch_types=[
            pltpu.VMEM((_CH,), jnp.int32),
            pltpu.VMEM((_CH, _D), jnp.float32),
            pltpu.SemaphoreType.DMA,
        ],
    )
    def k(x_hbm, slots_hbm, out_hbm, idx_v, rows_v, sem):
        wid = lax.axis_index("s") * 2 + lax.axis_index("c")
        base = wid * _RPW
        for ci in range(_RPW // _CH):
            off = base + ci * _CH
            pltpu.sync_copy(slots_hbm.at[pl.ds(off, _CH)], idx_v)
            pltpu.sync_copy(x_hbm.at[pl.ds(off, _CH)], rows_v)
            pltpu.async_copy(rows_v, out_hbm.at[idx_v], sem).wait()

    return k(x, slots)


def _gather_rows(ys, slots):
    @functools.partial(
        pl.kernel, mesh=_sc_mesh(),
        out_type=jax.ShapeDtypeStruct((_N, _O), jnp.float32),
        scratch_types=[
            pltpu.VMEM((_CH,), jnp.int32),
            pltpu.VMEM((_CH, _O), jnp.float32),
            pltpu.SemaphoreType.DMA,
        ],
    )
    def k(ys_hbm, slots_hbm, out_hbm, idx_v, rows_v, sem):
        wid = lax.axis_index("s") * 2 + lax.axis_index("c")
        base = wid * _RPW
        for ci in range(_RPW // _CH):
            off = base + ci * _CH
            pltpu.sync_copy(slots_hbm.at[pl.ds(off, _CH)], idx_v)
            pltpu.async_copy(ys_hbm.at[idx_v], rows_v, sem).wait()
            pltpu.sync_copy(rows_v, out_hbm.at[pl.ds(off, _CH)])

    return k(ys, slots)


def kernel(x, Wr, br, W1, b1, W2, b2):
    probs, ranks, counts = pl.pallas_call(
        _router_body,
        grid=(_NB,),
        in_specs=[
            pl.BlockSpec((_TB, _D), lambda b: (b, 0)),
            pl.BlockSpec((_D, _E), lambda b: (0, 0)),
            pl.BlockSpec((_E,), lambda b: (0,)),
        ],
        out_specs=[
            pl.BlockSpec((_TB, _E), lambda b: (b, 0)),
            pl.BlockSpec((_TB, _E), lambda b: (b, 0)),
            pl.BlockSpec((1, _E), lambda b: (0, 0)),
        ],
        out_shape=[
            jax.ShapeDtypeStruct((_N, _E), jnp.float32),
            jax.ShapeDtypeStruct((_N, _E), jnp.float32),
            jax.ShapeDtypeStruct((1, _E), jnp.float32),
        ],
        compiler_params=pltpu.CompilerParams(
            dimension_semantics=("arbitrary",)),
    )(x, Wr, br)

    slots2d, meta = pl.pallas_call(
        _slots_body,
        grid=(_NB,),
        in_specs=[
            pl.BlockSpec((_TB, _E), lambda b: (b, 0)),
            pl.BlockSpec((_TB, _E), lambda b: (b, 0)),
            pl.BlockSpec((1, _E), lambda b: (0, 0)),
        ],
        out_specs=[
            pl.BlockSpec((_TB, 1), lambda b: (b, 0)),
            pl.BlockSpec((8, _WPAD), lambda b: (0, 0)),
        ],
        out_shape=[
            jax.ShapeDtypeStruct((_N, 1), jnp.int32),
            jax.ShapeDtypeStruct((8, _WPAD), jnp.int32),
        ],
        compiler_params=pltpu.CompilerParams(
            dimension_semantics=("arbitrary",)),
    )(probs, ranks, counts)

    slots = slots2d.reshape(_N)
    xs = _scatter_rows(x, slots)

    grid_spec = pltpu.PrefetchScalarGridSpec(
        num_scalar_prefetch=1,
        grid=(_NW,),
        in_specs=[
            pl.BlockSpec((_MB, _D), lambda w, m: (m[0, w], 0)),
            pl.BlockSpec((1, _D, _F), lambda w, m: (m[1, w], 0, 0)),
            pl.BlockSpec((1, 1, _F), lambda w, m: (m[1, w], 0, 0)),
            pl.BlockSpec((1, _F, _O), lambda w, m: (m[1, w], 0, 0)),
            pl.BlockSpec((1, 1, _O), lambda w, m: (m[1, w], 0, 0)),
        ],
        out_specs=pl.BlockSpec((_MB, _O), lambda w, m: (m[0, w], 0)),
    )
    ys = pl.pallas_call(
        _mlp_body,
        grid_spec=grid_spec,
        out_shape=jax.ShapeDtypeStruct((_N, _O), jnp.float32),
        compiler_params=pltpu.CompilerParams(
            dimension_semantics=("arbitrary",)),
    )(meta, xs, W1, b1.reshape(_E, 1, _F), W2, b2.reshape(_E, 1, _O))

    out = _gather_rows(ys, slots)
    return out, probs, counts.reshape(_E)


# speedup vs baseline: 2.4457x; 1.0023x over previous
import functools

import jax
import jax.numpy as jnp
from jax import lax
from jax.experimental import pallas as pl
from jax.experimental.pallas import tpu as pltpu
from jax.experimental.pallas import tpu_sc as plsc

_E = 8
_D = 1024
_F = 2048
_O = 1024
_N = 4096

_TB = 512
_NB = _N // _TB
_MB = 256
_NT = _N // _MB
_NW = _NT + _E - 1
_WPAD = 32

_NWORK = 32
_RPW = _N // _NWORK
_CH = 64


def _router_body(x_ref, Wr_ref, br_ref, probs_ref, ranks_ref, counts_ref):
    b = pl.program_id(0)
    xb = x_ref[...]
    logits = jnp.dot(xb, Wr_ref[...], preferred_element_type=jnp.float32)
    logits = logits + br_ref[...][None, :]
    m = jnp.max(logits, axis=-1, keepdims=True)
    unn = jnp.exp(logits - m)
    probs = unn / jnp.sum(unn, axis=-1, keepdims=True)
    probs_ref[...] = probs

    routes = jnp.argmax(probs, axis=-1)
    iota_e = lax.broadcasted_iota(jnp.int32, (1, _E), 1)
    onehot = (routes[:, None] == iota_e).astype(jnp.float32)

    r = lax.broadcasted_iota(jnp.int32, (_TB, _TB), 0)
    c = lax.broadcasted_iota(jnp.int32, (_TB, _TB), 1)
    tril = (c < r).astype(jnp.float32)
    ranks_in = jnp.dot(tril, onehot, preferred_element_type=jnp.float32)

    prev = jnp.where(b == 0, 0.0, counts_ref[...])
    ranks_ref[...] = ranks_in + prev
    counts_ref[...] = prev + jnp.sum(onehot, axis=0, keepdims=True)


def _col_prefix(row, strict):
    a = jnp.broadcast_to(row, (_E, _E))
    r = lax.broadcasted_iota(jnp.int32, (_E, _E), 0)
    c = lax.broadcasted_iota(jnp.int32, (_E, _E), 1)
    sel = (c < r) if strict else (c <= r)
    return jnp.sum(jnp.where(sel, a, 0), axis=1, keepdims=True)


def _row_from_col(col):
    a = jnp.broadcast_to(col, (_E, _E))
    r = lax.broadcasted_iota(jnp.int32, (_E, _E), 0)
    c = lax.broadcasted_iota(jnp.int32, (_E, _E), 1)
    return jnp.sum(jnp.where(r == c, a, 0), axis=0, keepdims=True)


def _slots_body(probs_ref, ranks_ref, counts_ref, slots_ref, meta_ref):
    b = pl.program_id(0)
    counts_row = counts_ref[...]

    off_start_col = _col_prefix(counts_row, strict=True)
    off_start_row = _row_from_col(off_start_col)

    probs = probs_ref[...]
    routes = jnp.argmax(probs, axis=-1)
    iota_e = lax.broadcasted_iota(jnp.int32, (1, _E), 1)
    onehot = (routes[:, None] == iota_e).astype(jnp.float32)
    slot = jnp.sum(onehot * (off_start_row + ranks_ref[...]),
                   axis=1, keepdims=True)
    slots_ref[...] = slot.astype(jnp.int32)

    @pl.when(b == 0)
    def _meta():
        cnt_col = _col_prefix(counts_row, strict=False) - off_start_col
        cnt_i = cnt_col.astype(jnp.int32)
        os_i = off_start_col.astype(jnp.int32)
        oe_i = os_i + cnt_i
        ft = os_i // _MB
        lte = (oe_i + (_MB - 1)) // _MB
        n_tiles = jnp.where(cnt_i > 0, lte - ft, 0)

        nt_row = _row_from_col(n_tiles)
        item_start = _col_prefix(nt_row, strict=True)
        item_end = item_start + n_tiles
        n_items = jnp.sum(n_tiles)

        w = lax.broadcasted_iota(jnp.int32, (1, _WPAD), 1)
        m8 = (item_start <= w) & (w < item_end)
        e_iota = lax.broadcasted_iota(jnp.int32, (_E, _WPAD), 0)

        def sel(col):
            return jnp.sum(jnp.where(m8, jnp.broadcast_to(col, (_E, _WPAD)),
                                     0), axis=0, keepdims=True)

        e_of_w = jnp.sum(jnp.where(m8, e_iota, 0), axis=0, keepdims=True)
        j_of_w = w - sel(item_start)
        t_of_w = sel(ft) + j_of_w
        lo_w = jnp.maximum(sel(os_i), t_of_w * _MB)
        hi_w = jnp.minimum(sel(oe_i), (t_of_w + 1) * _MB)

        valid = w < n_items
        wi_tile = jnp.where(valid, t_of_w, _NT - 1)
        wi_expert = jnp.where(valid, e_of_w, _E - 1)
        wi_lo = jnp.where(valid, lo_w, _N)
        wi_hi = jnp.where(valid, hi_w, _N)
        zeros = jnp.zeros((4, _WPAD), jnp.int32)
        meta_ref[...] = jnp.concatenate(
            [wi_tile, wi_expert, wi_lo, wi_hi, zeros], axis=0)


def _mlp_body(meta_ref, xs_ref, W1_ref, b1_ref, W2_ref, b2_ref, ys_ref):
    w = pl.program_id(0)
    tile = meta_ref[0, w]
    lo = meta_ref[2, w]
    hi = meta_ref[3, w]
    rows = tile * _MB + lax.broadcasted_iota(jnp.int32, (_MB, 1), 0)
    mask = (rows >= lo) & (rows < hi)

    xb = xs_ref[...].astype(jnp.bfloat16)
    h = jnp.maximum(
        jnp.dot(xb, W1_ref[0].astype(jnp.bfloat16),
                preferred_element_type=jnp.float32)
        + b1_ref[0, 0][None, :], 0.0).astype(jnp.bfloat16)
    y = jnp.dot(h, W2_ref[0].astype(jnp.bfloat16),
                preferred_element_type=jnp.float32) \
        + b2_ref[0, 0][None, :]

    t_prev = jnp.where(w == 0, -1, meta_ref[0, jnp.maximum(w - 1, 0)])
    prev = jnp.where(tile != t_prev, 0.0, ys_ref[...])
    ys_ref[...] = jnp.where(mask, y, prev)


def _sc_mesh():
    return plsc.VectorSubcoreMesh(core_axis_name="c", subcore_axis_name="s")


def _scatter_rows(x, slots):
    @functools.partial(
        pl.kernel, mesh=_sc_mesh(),
        out_type=jax.ShapeDtypeStruct((_N, _D), jnp.float32),
        scratch_types=[
            pltpu.VMEM((_CH,), jnp.int32),
            pltpu.VMEM((_CH, _D), jnp.float32),
            pltpu.SemaphoreType.DMA,
        ],
    )
    def k(x_hbm, slots_hbm, out_hbm, idx_v, rows_v, sem):
        wid = lax.axis_index("s") * 2 + lax.axis_index("c")
        base = wid * _RPW
        for ci in range(_RPW // _CH):
            off = base + ci * _CH
            pltpu.sync_copy(slots_hbm.at[pl.ds(off, _CH)], idx_v)
            pltpu.sync_copy(x_hbm.at[pl.ds(off, _CH)], rows_v)
            pltpu.async_copy(rows_v, out_hbm.at[idx_v], sem).wait()

    return k(x, slots)


def _gather_rows(ys, slots):
    @functools.partial(
        pl.kernel, mesh=_sc_mesh(),
        out_type=jax.ShapeDtypeStruct((_N, _O), jnp.float32),
        scratch_types=[
            pltpu.VMEM((_CH,), jnp.int32),
            pltpu.VMEM((_CH, _O), jnp.float32),
            pltpu.SemaphoreType.DMA,
        ],
    )
    def k(ys_hbm, slots_hbm, out_hbm, idx_v, rows_v, sem):
        wid = lax.axis_index("s") * 2 + lax.axis_index("c")
        base = wid * _RPW
        for ci in range(_RPW // _CH):
            off = base + ci * _CH
            pltpu.sync_copy(slots_hbm.at[pl.ds(off, _CH)], idx_v)
            pltpu.async_copy(ys_hbm.at[idx_v], rows_v, sem).wait()
            pltpu.sync_copy(rows_v, out_hbm.at[pl.ds(off, _CH)])

    return k(ys, slots)


def kernel(x, Wr, br, W1, b1, W2, b2):
    probs, ranks, counts = pl.pallas_call(
        _router_body,
        grid=(_NB,),
        in_specs=[
            pl.BlockSpec((_TB, _D), lambda b: (b, 0)),
            pl.BlockSpec((_D, _E), lambda b: (0, 0)),
            pl.BlockSpec((_E,), lambda b: (0,)),
        ],
        out_specs=[
            pl.BlockSpec((_TB, _E), lambda b: (b, 0)),
            pl.BlockSpec((_TB, _E), lambda b: (b, 0)),
            pl.BlockSpec((1, _E), lambda b: (0, 0)),
        ],
        out_shape=[
            jax.ShapeDtypeStruct((_N, _E), jnp.float32),
            jax.ShapeDtypeStruct((_N, _E), jnp.float32),
            jax.ShapeDtypeStruct((1, _E), jnp.float32),
        ],
        compiler_params=pltpu.CompilerParams(
            dimension_semantics=("arbitrary",)),
    )(x, Wr, br)

    slots2d, meta = pl.pallas_call(
        _slots_body,
        grid=(_NB,),
        in_specs=[
            pl.BlockSpec((_TB, _E), lambda b: (b, 0)),
            pl.BlockSpec((_TB, _E), lambda b: (b, 0)),
            pl.BlockSpec((1, _E), lambda b: (0, 0)),
        ],
        out_specs=[
            pl.BlockSpec((_TB, 1), lambda b: (b, 0)),
            pl.BlockSpec((8, _WPAD), lambda b: (0, 0)),
        ],
        out_shape=[
            jax.ShapeDtypeStruct((_N, 1), jnp.int32),
            jax.ShapeDtypeStruct((8, _WPAD), jnp.int32),
        ],
        compiler_params=pltpu.CompilerParams(
            dimension_semantics=("arbitrary",)),
    )(probs, ranks, counts)

    slots = slots2d.reshape(_N)
    xs = _scatter_rows(x, slots)

    grid_spec = pltpu.PrefetchScalarGridSpec(
        num_scalar_prefetch=1,
        grid=(_NW,),
        in_specs=[
            pl.BlockSpec((_MB, _D), lambda w, m: (m[0, w], 0)),
            pl.BlockSpec((1, _D, _F), lambda w, m: (m[1, w], 0, 0)),
            pl.BlockSpec((1, 1, _F), lambda w, m: (m[1, w], 0, 0)),
            pl.BlockSpec((1, _F, _O), lambda w, m: (m[1, w], 0, 0)),
            pl.BlockSpec((1, 1, _O), lambda w, m: (m[1, w], 0, 0)),
        ],
        out_specs=pl.BlockSpec((_MB, _O), lambda w, m: (m[0, w], 0)),
    )
    ys = pl.pallas_call(
        _mlp_body,
        grid_spec=grid_spec,
        out_shape=jax.ShapeDtypeStruct((_N, _O), jnp.float32),
        compiler_params=pltpu.CompilerParams(
            dimension_semantics=("arbitrary",)),
    )(meta, xs, W1, b1.reshape(_E, 1, _F), W2, b2.reshape(_E, 1, _O))

    out = _gather_rows(ys, slots)
    return out, probs, counts.reshape(_E)
